# baseline (speedup 1.0000x reference)
import functools

import jax
import jax.numpy as jnp
from jax import lax
from jax.experimental import pallas as pl
from jax.experimental.pallas import tpu as pltpu
from jax.experimental.pallas import tpu_sc as plsc

N_PAD = 96
N_EDGE_CH = 94
NUM_CORES = 2
NUM_TILES = 16 * NUM_CORES
COLS_PT = 6400 // NUM_TILES
OFFS = tuple(list(range(0, COLS_PT - 15, 16))
             + ([COLS_PT - 16] if COLS_PT % 16 else []))


def _tanh(x):
    ax = jnp.abs(x)
    t = 1.0 - 2.0 / (jnp.exp(2.0 * ax) + 1.0)
    return jnp.sign(x) * t


def _rsqrt(d):
    bits = lax.bitcast_convert_type(d, jnp.int32)
    y = lax.bitcast_convert_type(
        jnp.int32(0x5F3759DF) - (bits >> 1), jnp.float32)
    half = 0.5 * d
    for _ in range(4):
        y = y * (1.5 - half * y * y)
    return y


def _sc_gcn(feature, edge_index, W1, b1, W2, b2, wfc, bfc):
    mesh = plsc.VectorSubcoreMesh(
        core_axis_name="c", subcore_axis_name="s", num_cores=NUM_CORES)

    @functools.partial(
        pl.kernel,
        mesh=mesh,
        out_type=jax.ShapeDtypeStruct((6400,), jnp.float32),
        compiler_params=pltpu.CompilerParams(
            use_tc_tiling_on_sc=False, needs_layout_passes=False),
        scratch_types=[
            pltpu.VMEM((94, 1), jnp.float32),
            pltpu.VMEM((2, 1504), jnp.int32),
            pltpu.VMEM((1, 4), jnp.float32),
            pltpu.VMEM((4,), jnp.float32),
            pltpu.VMEM((4, 1), jnp.float32),
            pltpu.VMEM((1,), jnp.float32),
            pltpu.VMEM((94, COLS_PT), jnp.float32),
            pltpu.VMEM((COLS_PT,), jnp.float32),
            pltpu.VMEM((N_PAD,), jnp.float32),
            pltpu.VMEM((N_PAD,), jnp.float32),
            pltpu.VMEM((N_PAD,), jnp.float32),
            pltpu.VMEM((N_PAD,), jnp.float32),
            pltpu.VMEM((N_PAD,), jnp.float32),
            pltpu.VMEM((N_PAD,), jnp.float32),
            pltpu.SemaphoreType.DMA,
            pltpu.SemaphoreType.DMA,
        ],
    )
    def k(feat_hbm, ed_hbm, w1_hbm, b1_hbm, w2_hbm, b2_hbm,
          wfc_hbm, bfc_hbm, out_hbm,
          fv, ed_v, w1_v, b1_v, w2_v, b2_v, wblk_v, obuf_v, dinv_v,
          s1_v, g1_v, g2_v, agg2_v, v_v, wsem, ssem):
        wid = lax.axis_index("s") * NUM_CORES + lax.axis_index("c")
        base = wid * COLS_PT

        wcp = pltpu.make_async_copy(
            wfc_hbm.at[:, pl.ds(base, COLS_PT)], wblk_v, wsem)
        wcp.start()
        cps = [
            pltpu.make_async_copy(ed_hbm, ed_v, ssem),
            pltpu.make_async_copy(feat_hbm, fv, ssem),
            pltpu.make_async_copy(w1_hbm, w1_v, ssem),
            pltpu.make_async_copy(b1_hbm, b1_v, ssem),
            pltpu.make_async_copy(w2_hbm, w2_v, ssem),
            pltpu.make_async_copy(b2_hbm, b2_v, ssem),
            pltpu.make_async_copy(
                bfc_hbm.at[pl.ds(base, COLS_PT)], obuf_v, ssem),
        ]
        for cp in cps:
            cp.start()
        for cp in cps:
            cp.wait()

        lane = lax.iota(jnp.int32, 16)
        zi = jnp.zeros((16,), jnp.int32)
        l3 = jnp.minimum(lane, 3)
        ones = jnp.ones((16,), jnp.float32)

        w1vec = plsc.load_gather(w1_v, [zi, l3])
        b1vec = plsc.load_gather(b1_v, [l3])
        w2vec = plsc.load_gather(w2_v, [l3, zi])
        b2vec = plsc.load_gather(b2_v, [zi])
        w10, w11, w12, w13 = w1vec[0], w1vec[1], w1vec[2], w1vec[3]
        b10, b11, b12, b13 = b1vec[0], b1vec[1], b1vec[2], b1vec[3]
        w20, w21, w22, w23 = w2vec[0], w2vec[1], w2vec[2], w2vec[3]
        b2s = b2vec[0]

        for i in range(N_PAD // 16):
            dinv_v[pl.ds(i * 16, 16)] = ones

        def deg_body(e, _):
            c = ed_v[1, pl.ds(e * 16, 16)]
            plsc.addupdate_scatter(dinv_v, [c], ones)
            return 0

        lax.fori_loop(0, N_EDGE_CH, deg_body, 0, unroll=4)

        for i in range(N_PAD // 16):
            sl = pl.ds(i * 16, 16)
            di = _rsqrt(dinv_v[sl])
            dinv_v[sl] = di
            fidx = jnp.minimum(lane + (i * 16), 93)
            g = di * plsc.load_gather(fv, [fidx, zi])
            g1_v[sl] = g
            s1_v[sl] = g

        def edge1_body(e, _):
            r = ed_v[0, pl.ds(e * 16, 16)]
            c = ed_v[1, pl.ds(e * 16, 16)]
            g = plsc.load_gather(g1_v, [r])
            plsc.addupdate_scatter(s1_v, [c], g)
            return 0

        lax.fori_loop(0, N_EDGE_CH, edge1_body, 0, unroll=4)

        for i in range(N_PAD // 16):
            sl = pl.ds(i * 16, 16)
            di = dinv_v[sl]
            m = s1_v[sl] * di
            h0 = _tanh(m * w10 + b10)
            h1 = _tanh(m * w11 + b11)
            h2 = _tanh(m * w12 + b12)
            h3 = _tanh(m * w13 + b13)
            x2 = h0 * w20 + h1 * w21 + h2 * w22 + h3 * w23
            g2 = di * x2
            g2_v[sl] = g2
            agg2_v[sl] = g2

        def edge2_body(e, _):
            r = ed_v[0, pl.ds(e * 16, 16)]
            c = ed_v[1, pl.ds(e * 16, 16)]
            g = plsc.load_gather(g2_v, [r])
            plsc.addupdate_scatter(agg2_v, [c], g)
            return 0

        lax.fori_loop(0, N_EDGE_CH, edge2_body, 0, unroll=4)

        for i in range(N_PAD // 16):
            sl = pl.ds(i * 16, 16)
            v_v[sl] = _tanh(agg2_v[sl] * dinv_v[sl] + b2s)

        wcp.wait()

        def mv_outer(i, accs):
            vvec = v_v[pl.ds(i * 16, 16)]
            nb = i * 16
            for l in range(16):
                s = vvec[l]
                accs = tuple(accs[j] + s * wblk_v[nb + l, pl.ds(OFFS[j], 16)]
                             for j in range(len(OFFS)))
            return accs

        init = tuple(obuf_v[pl.ds(o, 16)] for o in OFFS)
        accs = lax.fori_loop(0, 5, mv_outer, init)
        vtail = v_v[pl.ds(80, 16)]
        for l in range(14):
            s = vtail[l]
            accs = tuple(accs[j] + s * wblk_v[80 + l, pl.ds(OFFS[j], 16)]
                         for j in range(len(OFFS)))
        for j, o in enumerate(OFFS):
            obuf_v[pl.ds(o, 16)] = accs[j]
        pltpu.sync_copy(obuf_v, out_hbm.at[pl.ds(base, COLS_PT)])

    return k(feature, edge_index, W1, b1, W2, b2, wfc, bfc)


def kernel(feature, edge_index, W1, b1, W2, b2, Wfc, bfc):
    return _sc_gcn(feature, edge_index.astype(jnp.int32),
                   W1, b1, W2, b2, Wfc, bfc)


# speedup vs baseline: 1.0761x; 1.0761x over previous
import functools

import jax
import jax.numpy as jnp
from jax import lax
from jax.experimental import pallas as pl
from jax.experimental.pallas import tpu as pltpu
from jax.experimental.pallas import tpu_sc as plsc

N_PAD = 96
N_EDGE_CH = 94
NUM_CORES = 2
NUM_TILES = 16 * NUM_CORES
COLS_PT = 6400 // NUM_TILES
OFFS = tuple(list(range(0, COLS_PT - 15, 16))
             + ([COLS_PT - 16] if COLS_PT % 16 else []))


def _tanh(x):
    ax = jnp.abs(x)
    t = 1.0 - 2.0 / (jnp.exp(2.0 * ax) + 1.0)
    return jnp.sign(x) * t


def _rsqrt(d):
    bits = lax.bitcast_convert_type(d, jnp.int32)
    y = lax.bitcast_convert_type(
        jnp.int32(0x5F3759DF) - (bits >> 1), jnp.float32)
    half = 0.5 * d
    for _ in range(4):
        y = y * (1.5 - half * y * y)
    return y


def _sc_gcn(ed, fp, wfc, bfc):
    mesh = plsc.VectorSubcoreMesh(
        core_axis_name="c", subcore_axis_name="s", num_cores=NUM_CORES)

    @functools.partial(
        pl.kernel,
        mesh=mesh,
        out_type=jax.ShapeDtypeStruct((6400,), jnp.float32),
        compiler_params=pltpu.CompilerParams(
            use_tc_tiling_on_sc=False, needs_layout_passes=False,
            disable_bounds_checks=True, disable_semaphore_checks=True,
            skip_device_barrier=True),
        scratch_types=[
            pltpu.VMEM((3008,), jnp.int32),
            pltpu.VMEM((112,), jnp.float32),
            pltpu.VMEM((94, COLS_PT), jnp.float32),
            pltpu.VMEM((COLS_PT,), jnp.float32),
            pltpu.VMEM((N_PAD,), jnp.float32),
            pltpu.VMEM((N_PAD,), jnp.float32),
            pltpu.VMEM((N_PAD,), jnp.float32),
            pltpu.VMEM((N_PAD,), jnp.float32),
            pltpu.VMEM((N_PAD,), jnp.float32),
            pltpu.VMEM((N_PAD,), jnp.float32),
            pltpu.SemaphoreType.DMA,
            pltpu.SemaphoreType.DMA,
        ],
    )
    def k(ed_hbm, fp_hbm, wfc_hbm, bfc_hbm, out_hbm,
          ed_v, fp_v, wblk_v, obuf_v, dinv_v,
          s1_v, g1_v, g2_v, agg2_v, v_v, wsem, ssem):
        wid = lax.axis_index("s") * NUM_CORES + lax.axis_index("c")
        base = wid * COLS_PT

        wcp = pltpu.make_async_copy(
            wfc_hbm.at[:, pl.ds(base, COLS_PT)], wblk_v, wsem)
        wcp.start()
        cps = [
            pltpu.make_async_copy(ed_hbm, ed_v, ssem),
            pltpu.make_async_copy(fp_hbm, fp_v, ssem),
            pltpu.make_async_copy(
                bfc_hbm.at[pl.ds(base, COLS_PT)], obuf_v, ssem),
        ]
        for cp in cps:
            cp.start()
        for cp in cps:
            cp.wait()

        ones = jnp.ones((16,), jnp.float32)
        for i in range(N_PAD // 16):
            dinv_v[pl.ds(i * 16, 16)] = ones

        def deg_body(e, _):
            c = ed_v[pl.ds(1504 + e * 16, 16)]
            plsc.addupdate_scatter(dinv_v, [c], ones)
            return 0

        lax.fori_loop(0, N_EDGE_CH, deg_body, 0, unroll=4)

        pv = fp_v[pl.ds(96, 16)]
        w10, w11, w12, w13 = pv[0], pv[1], pv[2], pv[3]
        b10, b11, b12, b13 = pv[4], pv[5], pv[6], pv[7]
        w20, w21, w22, w23 = pv[8], pv[9], pv[10], pv[11]
        b2s = pv[12]

        for i in range(N_PAD // 16):
            sl = pl.ds(i * 16, 16)
            di = _rsqrt(dinv_v[sl])
            dinv_v[sl] = di
            g = di * fp_v[sl]
            g1_v[sl] = g
            s1_v[sl] = g

        def edge1_body(e, _):
            r = ed_v[pl.ds(e * 16, 16)]
            c = ed_v[pl.ds(1504 + e * 16, 16)]
            g = plsc.load_gather(g1_v, [r])
            plsc.addupdate_scatter(s1_v, [c], g)
            return 0

        lax.fori_loop(0, N_EDGE_CH, edge1_body, 0, unroll=4)

        for i in range(N_PAD // 16):
            sl = pl.ds(i * 16, 16)
            di = dinv_v[sl]
            m = s1_v[sl] * di
            h0 = _tanh(m * w10 + b10)
            h1 = _tanh(m * w11 + b11)
            h2 = _tanh(m * w12 + b12)
            h3 = _tanh(m * w13 + b13)
            x2 = h0 * w20 + h1 * w21 + h2 * w22 + h3 * w23
            g2 = di * x2
            g2_v[sl] = g2
            agg2_v[sl] = g2

        def edge2_body(e, _):
            r = ed_v[pl.ds(e * 16, 16)]
            c = ed_v[pl.ds(1504 + e * 16, 16)]
            g = plsc.load_gather(g2_v, [r])
            plsc.addupdate_scatter(agg2_v, [c], g)
            return 0

        lax.fori_loop(0, N_EDGE_CH, edge2_body, 0, unroll=4)

        for i in range(N_PAD // 16):
            sl = pl.ds(i * 16, 16)
            v_v[sl] = _tanh(agg2_v[sl] * dinv_v[sl] + b2s)

        wcp.wait()

        def mv_outer(i, accs):
            vvec = v_v[pl.ds(i * 16, 16)]
            nb = i * 16
            for l in range(16):
                s = vvec[l]
                accs = tuple(accs[j] + s * wblk_v[nb + l, pl.ds(OFFS[j], 16)]
                             for j in range(len(OFFS)))
            return accs

        init = tuple(obuf_v[pl.ds(o, 16)] for o in OFFS)
        accs = lax.fori_loop(0, 5, mv_outer, init)
        vtail = v_v[pl.ds(80, 16)]
        for l in range(14):
            s = vtail[l]
            accs = tuple(accs[j] + s * wblk_v[80 + l, pl.ds(OFFS[j], 16)]
                         for j in range(len(OFFS)))
        for j, o in enumerate(OFFS):
            obuf_v[pl.ds(o, 16)] = accs[j]
        pltpu.sync_copy(obuf_v, out_hbm.at[pl.ds(base, COLS_PT)])

    return k(ed, fp, wfc, bfc)


def kernel(feature, edge_index, W1, b1, W2, b2, Wfc, bfc):
    ed = edge_index.astype(jnp.int32).reshape(-1)
    feat = jnp.zeros((N_PAD,), jnp.float32).at[:94].set(feature[:, 0])
    params = jnp.concatenate([
        W1[0], b1, W2[:, 0], b2, jnp.zeros((3,), jnp.float32)])
    fp = jnp.concatenate([feat, params])
    return _sc_gcn(ed, fp, Wfc, bfc)


# speedup vs baseline: 1.0770x; 1.0008x over previous
import functools

import jax
import jax.numpy as jnp
from jax import lax
from jax.experimental import pallas as pl
from jax.experimental.pallas import tpu as pltpu
from jax.experimental.pallas import tpu_sc as plsc

N_PAD = 96
N_EDGE_CH = 94
NUM_CORES = 2
COLS_PT = 256
OFFS = tuple(range(0, COLS_PT, 16))


def _tanh(x):
    ax = jnp.abs(x)
    t = 1.0 - 2.0 / (jnp.exp(2.0 * ax) + 1.0)
    return jnp.sign(x) * t


def _rsqrt(d):
    bits = lax.bitcast_convert_type(d, jnp.int32)
    y = lax.bitcast_convert_type(
        jnp.int32(0x5F3759DF) - (bits >> 1), jnp.float32)
    half = 0.5 * d
    for _ in range(4):
        y = y * (1.5 - half * y * y)
    return y


def _sc_gcn(ed, fp, wfc, bfc):
    mesh = plsc.VectorSubcoreMesh(
        core_axis_name="c", subcore_axis_name="s", num_cores=NUM_CORES)

    @functools.partial(
        pl.kernel,
        mesh=mesh,
        out_type=jax.ShapeDtypeStruct((6400,), jnp.float32),
        compiler_params=pltpu.CompilerParams(needs_layout_passes=False),
        scratch_types=[
            pltpu.VMEM((3008,), jnp.int32),
            pltpu.VMEM((112,), jnp.float32),
            pltpu.VMEM((94, COLS_PT), jnp.float32),
            pltpu.VMEM((COLS_PT,), jnp.float32),
            pltpu.VMEM((N_PAD,), jnp.float32),
            pltpu.VMEM((N_PAD,), jnp.float32),
            pltpu.VMEM((N_PAD,), jnp.float32),
            pltpu.VMEM((N_PAD,), jnp.float32),
            pltpu.VMEM((N_PAD,), jnp.float32),
            pltpu.VMEM((N_PAD,), jnp.float32),
            pltpu.SemaphoreType.DMA,
            pltpu.SemaphoreType.DMA,
        ],
    )
    def k(ed_hbm, fp_hbm, wfc_hbm, bfc_hbm, out_hbm,
          ed_v, fp_v, wblk_v, obuf_v, dinv_v,
          s1_v, g1_v, g2_v, agg2_v, v_v, wsem, ssem):
        wid = lax.axis_index("s") * NUM_CORES + lax.axis_index("c")
        base = ((wid * 50) // 32) * 128

        wcp = pltpu.make_async_copy(
            wfc_hbm.at[:, pl.ds(base, COLS_PT)], wblk_v, wsem)
        wcp.start()
        cps = [
            pltpu.make_async_copy(ed_hbm, ed_v, ssem),
            pltpu.make_async_copy(fp_hbm, fp_v, ssem),
            pltpu.make_async_copy(
                bfc_hbm.at[pl.ds(base, COLS_PT)], obuf_v, ssem),
        ]
        for cp in cps:
            cp.start()
        for cp in cps:
            cp.wait()

        ones = jnp.ones((16,), jnp.float32)
        for i in range(N_PAD // 16):
            dinv_v[pl.ds(i * 16, 16)] = ones

        def deg_body(e, _):
            c = ed_v[pl.ds(1504 + e * 16, 16)]
            plsc.addupdate_scatter(dinv_v, [c], ones)
            return 0

        lax.fori_loop(0, N_EDGE_CH, deg_body, 0, unroll=4)

        pv = fp_v[pl.ds(96, 16)]
        w10, w11, w12, w13 = pv[0], pv[1], pv[2], pv[3]
        b10, b11, b12, b13 = pv[4], pv[5], pv[6], pv[7]
        w20, w21, w22, w23 = pv[8], pv[9], pv[10], pv[11]
        b2s = pv[12]

        for i in range(N_PAD // 16):
            sl = pl.ds(i * 16, 16)
            di = _rsqrt(dinv_v[sl])
            dinv_v[sl] = di
            g = di * fp_v[sl]
            g1_v[sl] = g
            s1_v[sl] = g

        def edge1_body(e, _):
            r = ed_v[pl.ds(e * 16, 16)]
            c = ed_v[pl.ds(1504 + e * 16, 16)]
            g = plsc.load_gather(g1_v, [r])
            plsc.addupdate_scatter(s1_v, [c], g)
            return 0

        lax.fori_loop(0, N_EDGE_CH, edge1_body, 0, unroll=4)

        for i in range(N_PAD // 16):
            sl = pl.ds(i * 16, 16)
            di = dinv_v[sl]
            m = s1_v[sl] * di
            h0 = _tanh(m * w10 + b10)
            h1 = _tanh(m * w11 + b11)
            h2 = _tanh(m * w12 + b12)
            h3 = _tanh(m * w13 + b13)
            x2 = h0 * w20 + h1 * w21 + h2 * w22 + h3 * w23
            g2 = di * x2
            g2_v[sl] = g2
            agg2_v[sl] = g2

        def edge2_body(e, _):
            r = ed_v[pl.ds(e * 16, 16)]
            c = ed_v[pl.ds(1504 + e * 16, 16)]
            g = plsc.load_gather(g2_v, [r])
            plsc.addupdate_scatter(agg2_v, [c], g)
            return 0

        lax.fori_loop(0, N_EDGE_CH, edge2_body, 0, unroll=4)

        for i in range(N_PAD // 16):
            sl = pl.ds(i * 16, 16)
            v_v[sl] = _tanh(agg2_v[sl] * dinv_v[sl] + b2s)

        wcp.wait()

        def mv_outer(i, accs):
            vvec = v_v[pl.ds(i * 16, 16)]
            nb = i * 16
            for l in range(16):
                s = vvec[l]
                accs = tuple(accs[j] + s * wblk_v[nb + l, pl.ds(OFFS[j], 16)]
                             for j in range(len(OFFS)))
            return accs

        init = tuple(obuf_v[pl.ds(o, 16)] for o in OFFS)
        accs = lax.fori_loop(0, 5, mv_outer, init)
        vtail = v_v[pl.ds(80, 16)]
        for l in range(14):
            s = vtail[l]
            accs = tuple(accs[j] + s * wblk_v[80 + l, pl.ds(OFFS[j], 16)]
                         for j in range(len(OFFS)))
        for j, o in enumerate(OFFS):
            obuf_v[pl.ds(o, 16)] = accs[j]
        pltpu.sync_copy(obuf_v, out_hbm.at[pl.ds(base, COLS_PT)])

    return k(ed, fp, wfc, bfc)


def kernel(feature, edge_index, W1, b1, W2, b2, Wfc, bfc):
    ed = edge_index.astype(jnp.int32).reshape(-1)
    feat = jnp.zeros((N_PAD,), jnp.float32).at[:94].set(feature[:, 0])
    params = jnp.concatenate([
        W1[0], b1, W2[:, 0], b2, jnp.zeros((3,), jnp.float32)])
    fp = jnp.concatenate([feat, params])
    return _sc_gcn(ed, fp, Wfc, bfc)
